# baseline (speedup 1.0000x reference)
import functools

import jax
import jax.numpy as jnp
from jax import lax
from jax.experimental import pallas as pl
from jax.experimental.pallas import tpu as pltpu
from jax.experimental.pallas import tpu_sc as plsc

_N = 10000
_E = 320000
_D = 128
_NC = 2
_NS = 16
_NW = _NC * _NS
_CHUNK = 128
_EPAD = 327680
_EPW = _EPAD // _NW
_NCHUNK = _EPW // _CHUNK
_HCH = _NCHUNK // 2
_NPAD = 10240
_RPT = _NPAD // _NS
_DEGW = 16
_BLK = 1000


@functools.lru_cache(maxsize=None)
def _sc_kernels():
    mesh = plsc.VectorSubcoreMesh(core_axis_name="c", subcore_axis_name="s")

    @functools.partial(
        pl.kernel,
        out_type=jax.ShapeDtypeStruct((_NC, _NPAD, _DEGW), jnp.float32),
        mesh=mesh,
        scratch_types=[
            pltpu.VMEM((_NCHUNK, _CHUNK), jnp.int32),
            pltpu.VMEM((_CHUNK,), jnp.float32),
            pltpu.VMEM((_RPT,), jnp.float32),
            pltpu.VMEM((_RPT, _DEGW), jnp.float32),
            pltpu.VMEM_SHARED((_NPAD,), jnp.float32),
        ],
    )
    def deg_kernel(dst_hbm, zeros_hbm, ones_hbm, out_hbm,
                   didx, ones, dloc, dbc, acc):
        cid = lax.axis_index("c")
        sid = lax.axis_index("s")
        wid = sid * _NC + cid
        rbase = sid * _RPT
        pltpu.sync_copy(zeros_hbm.at[pl.ds(rbase, _RPT)],
                        acc.at[pl.ds(rbase, _RPT)])
        pltpu.sync_copy(dst_hbm.at[wid], didx)
        pltpu.sync_copy(ones_hbm, ones)
        plsc.subcore_barrier()

        def body(j, carry):
            pltpu.sync_copy(ones, acc.at[didx.at[j]], add=True)
            return carry

        lax.fori_loop(0, _NCHUNK, body, 0)
        plsc.subcore_barrier()
        pltpu.sync_copy(acc.at[pl.ds(rbase, _RPT)], dloc)

        def bcast(t, carry):
            v = dloc[pl.ds(t * 16, 16)]
            for k in range(16):
                dbc[t * 16 + k, :] = jnp.full((_DEGW,), v[k], jnp.float32)
            return carry

        lax.fori_loop(0, _RPT // 16, bcast, 0)
        pltpu.sync_copy(dbc, out_hbm.at[cid, pl.ds(rbase, _RPT)])

    @functools.partial(
        pl.kernel,
        out_type=jax.ShapeDtypeStruct((_NC, _NPAD, _D), jnp.float32),
        mesh=mesh,
        scratch_types=[
            pltpu.VMEM((_HCH, _CHUNK), jnp.int32),
            pltpu.VMEM((_HCH, _CHUNK), jnp.int32),
            pltpu.VMEM((2, _CHUNK, _D), jnp.float32),
            pltpu.VMEM_SHARED((_NPAD, _D), jnp.float32),
            pltpu.SemaphoreType.DMA,
            pltpu.SemaphoreType.DMA,
            pltpu.SemaphoreType.DMA,
            pltpu.SemaphoreType.DMA,
        ],
    )
    def scatter_kernel(g_hbm, src_hbm, dst_hbm, zeros_hbm, out_hbm,
                       sidx, didx, rows, acc, gsem0, gsem1, ssem0, ssem1):
        cid = lax.axis_index("c")
        sid = lax.axis_index("s")
        wid = sid * _NC + cid
        rbase = sid * _RPT
        pltpu.sync_copy(zeros_hbm.at[pl.ds(rbase, _RPT)],
                        acc.at[pl.ds(rbase, _RPT)])
        plsc.subcore_barrier()

        def half(h, carry):
            pltpu.sync_copy(src_hbm.at[wid, pl.ds(h * _HCH, _HCH)], sidx)
            pltpu.sync_copy(dst_hbm.at[wid, pl.ds(h * _HCH, _HCH)], didx)
            pltpu.async_copy(g_hbm.at[sidx.at[0]], rows.at[0], gsem0)
            pltpu.async_copy(g_hbm.at[sidx.at[1]], rows.at[1], gsem1)

            def body(t, carry2):
                j0 = 2 * t
                j1 = 2 * t + 1
                pltpu.make_async_copy(g_hbm.at[sidx.at[j0]], rows.at[0],
                                      gsem0).wait()
                pltpu.async_copy(rows.at[0], acc.at[didx.at[j0]], ssem0,
                                 add=True)
                pltpu.make_async_copy(g_hbm.at[sidx.at[j1]], rows.at[1],
                                      gsem1).wait()
                pltpu.async_copy(rows.at[1], acc.at[didx.at[j1]], ssem1,
                                 add=True)
                pltpu.make_async_copy(rows.at[0], acc.at[didx.at[j0]],
                                      ssem0).wait()
                pltpu.make_async_copy(rows.at[1], acc.at[didx.at[j1]],
                                      ssem1).wait()

                @pl.when(t + 1 < _HCH // 2)
                def _():
                    pltpu.async_copy(g_hbm.at[sidx.at[j0 + 2]], rows.at[0],
                                     gsem0)
                    pltpu.async_copy(g_hbm.at[sidx.at[j1 + 2]], rows.at[1],
                                     gsem1)
                return carry2

            lax.fori_loop(0, _HCH // 2, body, 0)
            return carry

        lax.fori_loop(0, 2, half, 0)
        plsc.subcore_barrier()
        pltpu.sync_copy(acc.at[pl.ds(rbase, _RPT)],
                        out_hbm.at[cid, pl.ds(rbase, _RPT)])

    return deg_kernel, scatter_kernel


def _dinv_from(deg_ref):
    deg = deg_ref[0] + deg_ref[1] + 1.0
    return lax.rsqrt(deg)[:, 0:1]


def _tc_a1_body(x_ref, w1_ref, h1_ref):
    h1_ref[...] = jnp.dot(x_ref[...], w1_ref[...],
                          preferred_element_type=jnp.float32)


def _tc_a2_body(h1_ref, deg_ref, g1_ref):
    g1_ref[...] = _dinv_from(deg_ref) * h1_ref[...]


def _tc_b_body(acc_ref, g1_ref, deg_ref, b1_ref, w2_ref, g2_ref):
    dinv = _dinv_from(deg_ref)
    s = acc_ref[0] + acc_ref[1] + g1_ref[...]
    cur1 = jnp.maximum(dinv * s + b1_ref[...], 0.0)
    h2 = jnp.dot(cur1, w2_ref[...], preferred_element_type=jnp.float32)
    g2_ref[...] = dinv * h2


def _tc_c_body(acc_ref, g2_ref, deg_ref, b2_ref, hid_ref,
               wu_ref, bu_ref, wr_ref, br_ref, wc_ref, bc_ref, out_ref):
    dinv = _dinv_from(deg_ref)
    cur = dinv * (acc_ref[0] + acc_ref[1] + g2_ref[...]) + b2_ref[...]
    hid = hid_ref[...]

    def two_mm(w_ref, right):
        return (jnp.dot(cur, w_ref[0], preferred_element_type=jnp.float32)
                + jnp.dot(right, w_ref[1], preferred_element_type=jnp.float32))

    update = jax.nn.sigmoid(two_mm(wu_ref, hid) + bu_ref[...])
    reset = jax.nn.sigmoid(two_mm(wr_ref, hid) + br_ref[...])
    cand = jnp.tanh(two_mm(wc_ref, reset * hid) + bc_ref[...])
    out_ref[...] = hid + update * (cand - hid)


def _row_spec(blk):
    return pl.BlockSpec((blk, _D), lambda i: (i, 0))


_FULL_W = pl.BlockSpec((_D, _D), lambda i: (0, 0))
_FULL_W2 = pl.BlockSpec((2, _D, _D), lambda i: (0, 0, 0))
_BIAS = pl.BlockSpec((1, _D), lambda i: (0, 0))
_DEG_SPEC = pl.BlockSpec((_NC, _BLK, _DEGW), lambda i: (0, i, 0))
_ACC_SPEC = pl.BlockSpec((_NC, _BLK, _D), lambda i: (0, i, 0))


def _tc_a1(x, w1):
    return pl.pallas_call(
        _tc_a1_body,
        grid=(_N // _BLK,),
        in_specs=[_row_spec(_BLK), _FULL_W],
        out_specs=_row_spec(_BLK),
        out_shape=jax.ShapeDtypeStruct((_N, _D), jnp.float32),
    )(x, w1)


def _tc_a2(h1, degp):
    return pl.pallas_call(
        _tc_a2_body,
        grid=(_N // _BLK,),
        in_specs=[_row_spec(_BLK), _DEG_SPEC],
        out_specs=_row_spec(_BLK),
        out_shape=jax.ShapeDtypeStruct((_N, _D), jnp.float32),
    )(h1, degp)


def _tc_b(acc1, g1, degp, b1, w2):
    return pl.pallas_call(
        _tc_b_body,
        grid=(_N // _BLK,),
        in_specs=[_ACC_SPEC, _row_spec(_BLK), _DEG_SPEC, _BIAS, _FULL_W],
        out_specs=_row_spec(_BLK),
        out_shape=jax.ShapeDtypeStruct((_N, _D), jnp.float32),
    )(acc1, g1, degp, b1, w2)


def _tc_c(acc2, g2, degp, b2, hid, wu, bu, wr, br, wc, bc):
    return pl.pallas_call(
        _tc_c_body,
        grid=(_N // _BLK,),
        in_specs=[_ACC_SPEC, _row_spec(_BLK), _DEG_SPEC, _BIAS,
                  _row_spec(_BLK), _FULL_W2, _BIAS, _FULL_W2, _BIAS,
                  _FULL_W2, _BIAS],
        out_specs=_row_spec(_BLK),
        out_shape=jax.ShapeDtypeStruct((_N, _D), jnp.float32),
    )(acc2, g2, degp, b2, hid, wu, bu, wr, br, wc, bc)


def kernel(x, edge_index, hidden_state, W1, b1, W2, b2, Wu, bu, Wr, br, Wc, bc):
    deg_kernel, scatter_kernel = _sc_kernels()

    npad_e = _EPAD - _E
    pad_iota = jnp.arange(npad_e, dtype=edge_index.dtype)
    src3 = jnp.concatenate(
        [edge_index[0], pad_iota % _N]
    ).reshape(_NW, _NCHUNK, _CHUNK)
    dst3 = jnp.concatenate(
        [edge_index[1], _N + pad_iota % (_NPAD - _N)]
    ).reshape(_NW, _NCHUNK, _CHUNK)
    zeros_deg = jnp.zeros((_NPAD,), jnp.float32)
    zeros_nd = jnp.zeros((_NPAD, _D), jnp.float32)

    b1r = b1.reshape(1, _D)
    b2r = b2.reshape(1, _D)
    bur = bu.reshape(1, _D)
    brr = br.reshape(1, _D)
    bcr = bc.reshape(1, _D)
    wu2 = Wu.reshape(2, _D, _D)
    wr2 = Wr.reshape(2, _D, _D)
    wc2 = Wc.reshape(2, _D, _D)

    ones_ch = jnp.ones((_CHUNK,), jnp.float32)
    h1 = _tc_a1(x, W1)
    degp = deg_kernel(dst3, zeros_deg, ones_ch)
    g1 = _tc_a2(h1, degp)
    acc1 = scatter_kernel(g1, src3, dst3, zeros_nd)
    g2 = _tc_b(acc1, g1, degp, b1r, W2)
    acc2 = scatter_kernel(g2, src3, dst3, zeros_nd)
    return _tc_c(acc2, g2, degp, b2r, hidden_state,
                 wu2, bur, wr2, brr, wc2, bcr)


# speedup vs baseline: 1.2760x; 1.2760x over previous
import functools

import jax
import jax.numpy as jnp
from jax import lax
from jax.experimental import pallas as pl
from jax.experimental.pallas import tpu as pltpu
from jax.experimental.pallas import tpu_sc as plsc

_N = 10000
_E = 320000
_D = 128
_NC = 2
_NS = 16
_NW = _NC * _NS
_CHUNK = 128
_EPAD = 327680
_EPW = _EPAD // _NW
_NCHUNK = _EPW // _CHUNK
_HCH = _NCHUNK // 2
_NPAD = 10240
_RPT = _NPAD // _NS
_DEGW = 16
_BLK = 2000


@functools.lru_cache(maxsize=None)
def _sc_kernels():
    mesh = plsc.VectorSubcoreMesh(core_axis_name="c", subcore_axis_name="s")

    @functools.partial(
        pl.kernel,
        out_type=jax.ShapeDtypeStruct((_NC, _NPAD, _DEGW), jnp.float32),
        mesh=mesh,
        scratch_types=[
            pltpu.VMEM((_NCHUNK, _CHUNK), jnp.int32),
            pltpu.VMEM((_CHUNK,), jnp.float32),
            pltpu.VMEM((_RPT,), jnp.float32),
            pltpu.VMEM((_RPT, _DEGW), jnp.float32),
            pltpu.VMEM_SHARED((_NPAD,), jnp.float32),
        ],
    )
    def deg_kernel(dst_hbm, zeros_hbm, ones_hbm, out_hbm,
                   didx, ones, dloc, dbc, acc):
        cid = lax.axis_index("c")
        sid = lax.axis_index("s")
        wid = sid * _NC + cid
        rbase = sid * _RPT
        pltpu.sync_copy(zeros_hbm.at[pl.ds(rbase, _RPT)],
                        acc.at[pl.ds(rbase, _RPT)])
        pltpu.sync_copy(dst_hbm.at[wid], didx)
        pltpu.sync_copy(ones_hbm, ones)
        plsc.subcore_barrier()

        def body(j, carry):
            pltpu.sync_copy(ones, acc.at[didx.at[j]], add=True)
            return carry

        lax.fori_loop(0, _NCHUNK, body, 0)
        plsc.subcore_barrier()
        pltpu.sync_copy(acc.at[pl.ds(rbase, _RPT)], dloc)

        def bcast(t, carry):
            v = dloc[pl.ds(t * 16, 16)]
            for k in range(16):
                dbc[t * 16 + k, :] = jnp.full((_DEGW,), v[k], jnp.float32)
            return carry

        lax.fori_loop(0, _RPT // 16, bcast, 0)
        pltpu.sync_copy(dbc, out_hbm.at[cid, pl.ds(rbase, _RPT)])

    @functools.partial(
        pl.kernel,
        out_type=jax.ShapeDtypeStruct((_NC, _NPAD, _D), jnp.float32),
        mesh=mesh,
        scratch_types=[
            pltpu.VMEM((_HCH, _CHUNK), jnp.int32),
            pltpu.VMEM((_HCH, _CHUNK), jnp.int32),
            pltpu.VMEM((2, _CHUNK, _D), jnp.float32),
            pltpu.VMEM_SHARED((_NPAD, _D), jnp.float32),
            pltpu.SemaphoreType.DMA,
            pltpu.SemaphoreType.DMA,
            pltpu.SemaphoreType.DMA,
            pltpu.SemaphoreType.DMA,
        ],
    )
    def scatter_kernel(g_hbm, src_hbm, dst_hbm, zeros_hbm, out_hbm,
                       sidx, didx, rows, acc, gsem0, gsem1, ssem0, ssem1):
        cid = lax.axis_index("c")
        sid = lax.axis_index("s")
        wid = sid * _NC + cid
        rbase = sid * _RPT
        pltpu.sync_copy(zeros_hbm.at[pl.ds(rbase, _RPT)],
                        acc.at[pl.ds(rbase, _RPT)])
        plsc.subcore_barrier()

        def half(h, carry):
            pltpu.sync_copy(src_hbm.at[wid, pl.ds(h * _HCH, _HCH)], sidx)
            pltpu.sync_copy(dst_hbm.at[wid, pl.ds(h * _HCH, _HCH)], didx)
            pltpu.async_copy(g_hbm.at[sidx.at[0]], rows.at[0], gsem0)

            def body(t, carry2):
                j0 = 2 * t
                j1 = 2 * t + 1
                pltpu.async_copy(g_hbm.at[sidx.at[j1]], rows.at[1], gsem1)
                pltpu.make_async_copy(g_hbm.at[sidx.at[j0]], rows.at[0],
                                      gsem0).wait()
                pltpu.sync_copy(rows.at[0], acc.at[didx.at[j0]], add=True)

                @pl.when(t + 1 < _HCH // 2)
                def _():
                    pltpu.async_copy(g_hbm.at[sidx.at[j0 + 2]], rows.at[0],
                                     gsem0)

                pltpu.make_async_copy(g_hbm.at[sidx.at[j1]], rows.at[1],
                                      gsem1).wait()
                pltpu.sync_copy(rows.at[1], acc.at[didx.at[j1]], add=True)
                return carry2

            lax.fori_loop(0, _HCH // 2, body, 0)
            return carry

        lax.fori_loop(0, 2, half, 0)
        plsc.subcore_barrier()
        pltpu.sync_copy(acc.at[pl.ds(rbase, _RPT)],
                        out_hbm.at[cid, pl.ds(rbase, _RPT)])

    return deg_kernel, scatter_kernel


def _dinv_from(deg_ref):
    deg = deg_ref[0] + deg_ref[1] + 1.0
    return lax.rsqrt(deg)[:, 0:1]


def _tc_a1_body(x_ref, w1_ref, h1_ref):
    h1_ref[...] = jnp.dot(x_ref[...], w1_ref[...],
                          preferred_element_type=jnp.float32)


def _tc_a2_body(h1_ref, deg_ref, g1_ref):
    g1_ref[...] = _dinv_from(deg_ref) * h1_ref[...]


def _tc_b_body(acc_ref, g1_ref, deg_ref, b1_ref, w2_ref, g2_ref):
    dinv = _dinv_from(deg_ref)
    s = acc_ref[0] + acc_ref[1] + g1_ref[...]
    cur1 = jnp.maximum(dinv * s + b1_ref[...], 0.0)
    h2 = jnp.dot(cur1, w2_ref[...], preferred_element_type=jnp.float32)
    g2_ref[...] = dinv * h2


def _tc_c_body(acc_ref, g2_ref, deg_ref, b2_ref, hid_ref,
               wu_ref, bu_ref, wr_ref, br_ref, wc_ref, bc_ref, out_ref):
    dinv = _dinv_from(deg_ref)
    cur = dinv * (acc_ref[0] + acc_ref[1] + g2_ref[...]) + b2_ref[...]
    hid = hid_ref[...]

    def two_mm(w_ref, right):
        return (jnp.dot(cur, w_ref[0], preferred_element_type=jnp.float32)
                + jnp.dot(right, w_ref[1], preferred_element_type=jnp.float32))

    update = jax.nn.sigmoid(two_mm(wu_ref, hid) + bu_ref[...])
    reset = jax.nn.sigmoid(two_mm(wr_ref, hid) + br_ref[...])
    cand = jnp.tanh(two_mm(wc_ref, reset * hid) + bc_ref[...])
    out_ref[...] = hid + update * (cand - hid)


def _row_spec(blk):
    return pl.BlockSpec((blk, _D), lambda i: (i, 0))


_FULL_W = pl.BlockSpec((_D, _D), lambda i: (0, 0))
_FULL_W2 = pl.BlockSpec((2, _D, _D), lambda i: (0, 0, 0))
_BIAS = pl.BlockSpec((1, _D), lambda i: (0, 0))
_DEG_SPEC = pl.BlockSpec((_NC, _BLK, _DEGW), lambda i: (0, i, 0))
_ACC_SPEC = pl.BlockSpec((_NC, _BLK, _D), lambda i: (0, i, 0))


def _tc_a1(x, w1):
    return pl.pallas_call(
        _tc_a1_body,
        grid=(_N // _BLK,),
        in_specs=[_row_spec(_BLK), _FULL_W],
        out_specs=_row_spec(_BLK),
        out_shape=jax.ShapeDtypeStruct((_N, _D), jnp.float32),
    )(x, w1)


def _tc_a2(h1, degp):
    return pl.pallas_call(
        _tc_a2_body,
        grid=(_N // _BLK,),
        in_specs=[_row_spec(_BLK), _DEG_SPEC],
        out_specs=_row_spec(_BLK),
        out_shape=jax.ShapeDtypeStruct((_N, _D), jnp.float32),
    )(h1, degp)


def _tc_b(acc1, g1, degp, b1, w2):
    return pl.pallas_call(
        _tc_b_body,
        grid=(_N // _BLK,),
        in_specs=[_ACC_SPEC, _row_spec(_BLK), _DEG_SPEC, _BIAS, _FULL_W],
        out_specs=_row_spec(_BLK),
        out_shape=jax.ShapeDtypeStruct((_N, _D), jnp.float32),
    )(acc1, g1, degp, b1, w2)


def _tc_c(acc2, g2, degp, b2, hid, wu, bu, wr, br, wc, bc):
    return pl.pallas_call(
        _tc_c_body,
        grid=(_N // _BLK,),
        in_specs=[_ACC_SPEC, _row_spec(_BLK), _DEG_SPEC, _BIAS,
                  _row_spec(_BLK), _FULL_W2, _BIAS, _FULL_W2, _BIAS,
                  _FULL_W2, _BIAS],
        out_specs=_row_spec(_BLK),
        out_shape=jax.ShapeDtypeStruct((_N, _D), jnp.float32),
    )(acc2, g2, degp, b2, hid, wu, bu, wr, br, wc, bc)


def kernel(x, edge_index, hidden_state, W1, b1, W2, b2, Wu, bu, Wr, br, Wc, bc):
    deg_kernel, scatter_kernel = _sc_kernels()

    npad_e = _EPAD - _E
    pad_iota = jnp.arange(npad_e, dtype=edge_index.dtype)
    src3 = jnp.concatenate(
        [edge_index[0], pad_iota % _N]
    ).reshape(_NW, _NCHUNK, _CHUNK)
    dst3 = jnp.concatenate(
        [edge_index[1], _N + pad_iota % (_NPAD - _N)]
    ).reshape(_NW, _NCHUNK, _CHUNK)
    zeros_deg = jnp.zeros((_NPAD,), jnp.float32)
    zeros_nd = jnp.zeros((_NPAD, _D), jnp.float32)

    b1r = b1.reshape(1, _D)
    b2r = b2.reshape(1, _D)
    bur = bu.reshape(1, _D)
    brr = br.reshape(1, _D)
    bcr = bc.reshape(1, _D)
    wu2 = Wu.reshape(2, _D, _D)
    wr2 = Wr.reshape(2, _D, _D)
    wc2 = Wc.reshape(2, _D, _D)

    ones_ch = jnp.ones((_CHUNK,), jnp.float32)
    h1 = _tc_a1(x, W1)
    degp = deg_kernel(dst3, zeros_deg, ones_ch)
    g1 = _tc_a2(h1, degp)
    acc1 = scatter_kernel(g1, src3, dst3, zeros_nd)
    g2 = _tc_b(acc1, g1, degp, b1r, W2)
    acc2 = scatter_kernel(g2, src3, dst3, zeros_nd)
    return _tc_c(acc2, g2, degp, b2r, hidden_state,
                 wu2, bur, wr2, brr, wc2, bcr)
